# Initial kernel scaffold; baseline (speedup 1.0000x reference)
import jax
import jax.numpy as jnp
from jax.experimental import pallas as pl


def kernel(input_words, emb, W1, b1, g1, be1, m1, v1, W2, b2, g2, be2, m2, v2, Wc, bc, gc, bec, mc, vc):
    raise NotImplementedError("write your pallas kernel here")


# speedup vs baseline: 38.0294x; 38.0294x over previous
import functools

import jax
import jax.numpy as jnp
from jax import lax
from jax.experimental import pallas as pl
from jax.experimental.pallas import tpu as pltpu
from jax.experimental.pallas import tpu_sc as plsc

_B, _L = 1024, 200
_V, _D, _H, _C = 1000000, 128, 512, 1000

_NC, _NS = 2, 16
_NW = _NC * _NS
_BPW = _B // _NW
_HALF = _L // 2
_NH = 2 * _BPW
_LANES = 16
_DC = _D // _LANES


def _pool_body(words_hbm, emb_hbm, out_hbm, idx_v, rows0, rows1, acc_v,
               sem0, sem1):
    wid = lax.axis_index("s") * _NC + lax.axis_index("c")
    base = wid * _BPW
    pltpu.sync_copy(words_hbm.at[pl.ds(base * 2, _NH)], idx_v)

    rows = (rows0, rows1)
    sems = (sem0, sem1)

    def start(j, buf):
        pltpu.async_copy(emb_hbm.at[idx_v.at[j]], rows[buf], sems[buf])

    def wait(j, buf):
        pltpu.make_async_copy(emb_hbm.at[idx_v.at[j]], rows[buf],
                              sems[buf]).wait()

    def reduce_half(j, buf, h):
        wait(j, buf)
        r = rows[buf]
        accs = tuple(r[0, pl.ds(c * _LANES, _LANES)] for c in range(_DC))

        def body(l, accs):
            return tuple(jnp.maximum(a, r[l, pl.ds(c * _LANES, _LANES)])
                         for c, a in enumerate(accs))

        accs = lax.fori_loop(1, _HALF, body, accs, unroll=4)
        s = j // 2
        if h == 0:
            for c in range(_DC):
                acc_v[s, pl.ds(c * _LANES, _LANES)] = accs[c]
        else:
            for c in range(_DC):
                acc_v[s, pl.ds(c * _LANES, _LANES)] = jnp.maximum(
                    acc_v[s, pl.ds(c * _LANES, _LANES)], accs[c])

    start(0, 0)
    start(1, 1)

    @pl.loop(0, _NH - 4, step=2)
    def _(j):
        for b in range(2):
            reduce_half(j + b, b, b)
            start(j + b + 2, b)

    reduce_half(_NH - 4, 0, 0)
    start(_NH - 2, 0)
    reduce_half(_NH - 3, 1, 1)
    start(_NH - 1, 1)
    reduce_half(_NH - 2, 0, 0)
    reduce_half(_NH - 1, 1, 1)

    pltpu.sync_copy(acc_v, out_hbm.at[pl.ds(base, _BPW)])


@jax.jit
def _pool(words2, emb):
    mesh = plsc.VectorSubcoreMesh(core_axis_name="c", subcore_axis_name="s")
    f = pl.kernel(
        _pool_body,
        out_type=jax.ShapeDtypeStruct((_B, _D), jnp.float32),
        mesh=mesh,
        scratch_types=[
            pltpu.VMEM((_NH, _HALF), jnp.int32),
            pltpu.VMEM((_HALF, _D), jnp.float32),
            pltpu.VMEM((_HALF, _D), jnp.float32),
            pltpu.VMEM((_BPW, _D), jnp.float32),
            pltpu.SemaphoreType.DMA,
            pltpu.SemaphoreType.DMA,
        ],
    )
    return f(words2, emb)


_BT = 256
_CP = 1024


def _mlp_body(x_ref, w1_ref, w2_ref, wc_ref,
              b1_ref, g1_ref, be1_ref, m1_ref, v1_ref,
              b2_ref, g2_ref, be2_ref, m2_ref, v2_ref,
              bc_ref, gc_ref, bec_ref, mc_ref, vc_ref,
              out_ref):
    def bn(h, g, be, m, v):
        return (h - m[...]) / jnp.sqrt(v[...] + 1e-5) * g[...] + be[...]

    def elu(h):
        return jnp.where(h > 0, h, jnp.exp(jnp.minimum(h, 0.0)) - 1.0)

    h = jnp.dot(x_ref[...], w1_ref[...], preferred_element_type=jnp.float32)
    h = elu(bn(h + b1_ref[...], g1_ref, be1_ref, m1_ref, v1_ref))
    h = jnp.dot(h, w2_ref[...], preferred_element_type=jnp.float32)
    h = elu(bn(h + b2_ref[...], g2_ref, be2_ref, m2_ref, v2_ref))
    o = jnp.dot(h, wc_ref[...], preferred_element_type=jnp.float32)
    out_ref[...] = bn(o + bc_ref[...], gc_ref, bec_ref, mc_ref, vc_ref)


@jax.jit
def _mlp(x, w1t, w2t, wct, *vecs):
    full = lambda shape: pl.BlockSpec(shape, lambda i: (0, 0))
    vec_specs = [full(v.shape) for v in vecs]
    return pl.pallas_call(
        _mlp_body,
        grid=(_B // _BT,),
        in_specs=[pl.BlockSpec((_BT, _D), lambda i: (i, 0)),
                  full((_D, _H)), full((_H, _H)), full((_H, _CP))]
                 + vec_specs,
        out_specs=pl.BlockSpec((_BT, _CP), lambda i: (i, 0)),
        out_shape=jax.ShapeDtypeStruct((_B, _CP), jnp.float32),
    )(x, w1t, w2t, wct, *vecs)


def kernel(input_words, emb, W1, b1, g1, be1, m1, v1,
           W2, b2, g2, be2, m2, v2, Wc, bc, gc, bec, mc, vc):
    words2 = input_words.astype(jnp.int32).reshape(2 * _B, _HALF)
    pooled = _pool(words2, emb)

    pad = _CP - _C
    row = lambda v: v.reshape(1, -1)
    padr = lambda v, c=0.0: jnp.pad(v, (0, pad), constant_values=c).reshape(1, -1)
    out = _mlp(
        pooled, W1.T, W2.T,
        jnp.pad(Wc, ((0, pad), (0, 0))).T,
        row(b1), row(g1), row(be1), row(m1), row(v1),
        row(b2), row(g2), row(be2), row(m2), row(v2),
        padr(bc), padr(gc, 1.0), padr(bec), padr(mc), padr(vc, 1.0),
    )
    return out[:, :_C]
